# baseline (speedup 1.0000x reference)
import functools

import jax
import jax.numpy as jnp
from jax import lax
from jax.experimental import pallas as pl
from jax.experimental.pallas import tpu as pltpu
from jax.experimental.pallas import tpu_sc as plsc

N_NODES = 10000
N_EDGES = 320000
D = 128

NC = 2
NS = 16
NW = NC * NS
CHUNK = 112
N_CHUNKS = 90
E_PER_W = CHUNK * N_CHUNKS
E_TOTAL = E_PER_W * NW
N_PAD = 10112
ROWS_PER_S = N_PAD // NS
NBUF = 3
NIDX = 4


def _sc_aggregate(h, idx, zeros):
    mesh = plsc.VectorSubcoreMesh(core_axis_name="c", subcore_axis_name="s")

    @functools.partial(
        pl.kernel,
        out_type=jax.ShapeDtypeStruct((NC, N_PAD, D), jnp.float32),
        mesh=mesh,
        scratch_types=[
            pltpu.VMEM((NIDX, 2, CHUNK), jnp.int32),
            pltpu.VMEM((NBUF, CHUNK, D), jnp.float32),
            pltpu.VMEM_SHARED((N_PAD, D), jnp.float32),
            pltpu.SemaphoreType.DMA,
            pltpu.SemaphoreType.DMA,
            pltpu.SemaphoreType.DMA,
        ],
    )
    def agg(h_hbm, idx_hbm, zeros_hbm, out_hbm,
            idx_v, rows_v, acc, sem_i, sem_g, sem_s):
        c = lax.axis_index("c")
        s = lax.axis_index("s")
        wid = c * NS + s

        pltpu.sync_copy(zeros_hbm.at[pl.ds(s * ROWS_PER_S, ROWS_PER_S)],
                        acc.at[pl.ds(s * ROWS_PER_S, ROWS_PER_S)])
        plsc.subcore_barrier()

        def idx_copy(j):
            return pltpu.make_async_copy(
                idx_hbm.at[wid, j], idx_v.at[lax.rem(j, NIDX)], sem_i)

        def gather(j):
            return pltpu.make_async_copy(
                h_hbm.at[idx_v.at[lax.rem(j, NIDX), 0]],
                rows_v.at[lax.rem(j, NBUF)], sem_g)

        def scatter(j):
            return pltpu.make_async_copy(
                rows_v.at[lax.rem(j, NBUF)],
                acc.at[idx_v.at[lax.rem(j, NIDX), 1]], sem_s)

        pltpu.sync_copy(idx_hbm.at[wid, 0], idx_v.at[0])
        pltpu.sync_copy(idx_hbm.at[wid, 1], idx_v.at[1])
        gather(0).start()

        def body(i, carry):
            @pl.when(i >= 2)
            def _():
                scatter(i - 2).wait()

            @pl.when(jnp.logical_and(i + 1 >= 2, i + 1 < N_CHUNKS))
            def _():
                idx_copy(i + 1).wait()

            @pl.when(i + 2 < N_CHUNKS)
            def _():
                idx_copy(i + 2).start()

            gather(i).wait()

            @pl.when(i + 1 < N_CHUNKS)
            def _():
                gather(i + 1).start()

            pltpu.async_copy(rows_v.at[lax.rem(i, NBUF)],
                             acc.at[idx_v.at[lax.rem(i, NIDX), 1]],
                             sem_s, add=True)
            return carry

        lax.fori_loop(0, N_CHUNKS, body, 0)
        scatter(N_CHUNKS - 2).wait()
        scatter(N_CHUNKS - 1).wait()
        plsc.subcore_barrier()

        pltpu.sync_copy(acc.at[pl.ds(s * ROWS_PER_S, ROWS_PER_S)],
                        out_hbm.at[c, pl.ds(s * ROWS_PER_S, ROWS_PER_S)])

    return agg(h, idx, zeros)


def _tc_linear(partials, W, b):
    BLK = 400
    grid = N_NODES // BLK

    def body(p0_ref, p1_ref, w_ref, b_ref, out_ref):
        ah = p0_ref[0] + p1_ref[0]
        out_ref[...] = jnp.maximum(
            jnp.dot(ah, w_ref[...], preferred_element_type=jnp.float32)
            + b_ref[...], 0.0)

    return pl.pallas_call(
        body,
        grid=(grid,),
        in_specs=[
            pl.BlockSpec((1, BLK, D), lambda i: (0, i, 0)),
            pl.BlockSpec((1, BLK, D), lambda i: (1, i, 0)),
            pl.BlockSpec((D, D), lambda i: (0, 0)),
            pl.BlockSpec((1, D), lambda i: (0, 0)),
        ],
        out_specs=pl.BlockSpec((BLK, D), lambda i: (i, 0)),
        out_shape=jax.ShapeDtypeStruct((N_NODES, D), jnp.float32),
    )(partials, partials, W, b)


def kernel(h, edge_index, W, b):
    ei = edge_index.astype(jnp.int32)
    n_dummy = E_TOTAL - N_EDGES
    src = jnp.concatenate([ei[0], jnp.zeros((n_dummy,), jnp.int32)])
    dst = jnp.concatenate(
        [ei[1], jnp.full((n_dummy,), N_NODES, jnp.int32)])
    idx = jnp.stack([src.reshape(NW, N_CHUNKS, CHUNK),
                     dst.reshape(NW, N_CHUNKS, CHUNK)], axis=2)
    zeros = jnp.zeros((N_PAD, D), jnp.float32)
    partials = _sc_aggregate(h, idx, zeros)
    return _tc_linear(partials, W, b.reshape(1, D))
